# Initial kernel scaffold; baseline (speedup 1.0000x reference)
import jax
import jax.numpy as jnp
from jax.experimental import pallas as pl


def kernel(z, codebook):
    raise NotImplementedError("write your pallas kernel here")


# speedup vs baseline: 1.2925x; 1.2925x over previous
import functools

import jax
import jax.numpy as jnp
from jax.experimental import pallas as pl
from jax.experimental.pallas import tpu as pltpu

_BM = 256
_BK = 1024


def _vq_loss_kernel(z_ref, ct_ref, out_ref, *, n_tokens, k_codes):
    zb = z_ref[:]

    def body(i, minv):
        cb = ct_ref[:, pl.ds(i * _BK, _BK)]
        dots = jnp.dot(zb, cb, preferred_element_type=jnp.float32)
        csq = jnp.sum(cb * cb, axis=0)
        part = csq[None, :] - 2.0 * dots
        return jnp.minimum(minv, jnp.min(part, axis=1, keepdims=True))

    minv = jax.lax.fori_loop(
        0, k_codes // _BK, body,
        jnp.full((zb.shape[0], 1), jnp.inf, dtype=jnp.float32))
    zsq = jnp.sum(zb * zb, axis=1, keepdims=True)
    s = jnp.sum(minv + zsq)

    @pl.when(pl.program_id(0) == 0)
    def _init():
        out_ref[0, 0] = 0.0

    out_ref[0, 0] += s / n_tokens


def kernel(z, codebook):
    n, d = z.shape
    k = codebook.shape[0]
    ct = codebook.T
    out = pl.pallas_call(
        functools.partial(_vq_loss_kernel, n_tokens=n, k_codes=k),
        grid=(n // _BM,),
        in_specs=[
            pl.BlockSpec((_BM, d), lambda m: (m, 0)),
            pl.BlockSpec((d, k), lambda m: (0, 0)),
        ],
        out_specs=pl.BlockSpec(memory_space=pltpu.SMEM),
        out_shape=jax.ShapeDtypeStruct((1, 1), jnp.float32),
    )(z, ct)
    return out[0, 0]
